# baseline (speedup 1.0000x reference)
import functools

import jax
import jax.numpy as jnp
from jax import lax
from jax.experimental import pallas as pl
from jax.experimental.pallas import tpu as pltpu
from jax.experimental.pallas import tpu_sc as plsc

B, C, H, W = 4, 96, 224, 224
HW = H * W
N = B * HW
PAD = W + 1
NW = 32
CH = 64
TPW = N // NW
NCH = TPW // CH
GRP = CH // 16
UNROLL = 7
NMB = 4
NRB = 2

_mesh = plsc.VectorSubcoreMesh(core_axis_name="c", subcore_axis_name="s")

_meta_scratch = [
    pltpu.VMEM((CH,), jnp.int32),
    pltpu.VMEM((CH,), jnp.int32),
    pltpu.VMEM((CH,), jnp.int32),
    pltpu.VMEM((CH,), jnp.int32),
    pltpu.VMEM((CH,), jnp.int32),
    pltpu.VMEM((CH,), jnp.float32),
    pltpu.VMEM((CH,), jnp.float32),
    pltpu.VMEM((CH,), jnp.float32),
    pltpu.VMEM((CH,), jnp.float32),
]
_row_scratch = [pltpu.VMEM((CH, C), jnp.float32)] * 4


@functools.partial(
    pl.kernel,
    mesh=_mesh,
    compiler_params=pltpu.CompilerParams(use_tc_tiling_on_sc=False),
    out_type=jax.ShapeDtypeStruct((N, C), jnp.float32),
    scratch_types=(
        _meta_scratch * NMB
        + _row_scratch * NRB
        + [pltpu.VMEM((CH, C), jnp.float32)] * NRB
        + [pltpu.SemaphoreType.DMA] * (NMB + NRB + NRB)
    ),
)
def _sc_gather(xt_hbm, nb_hbm, v0_hbm, v1_hbm, v2_hbm, v3_hbm, sidx_hbm,
               out_hbm, *scr):
    mb = [scr[9 * k:9 * (k + 1)] for k in range(NMB)]
    o = 9 * NMB
    rb = [scr[o + 4 * k:o + 4 * (k + 1)] for k in range(NRB)]
    o += 4 * NRB
    ob = scr[o:o + NRB]
    o += NRB
    msem = scr[o:o + NMB]
    o += NMB
    rsem = scr[o:o + NRB]
    o += NRB
    osem = scr[o:o + NRB]

    wid = lax.axis_index("s") * 2 + lax.axis_index("c")
    wbase = wid * TPW

    def start_meta(gi, k):
        sidx_v, nbv = mb[k][0], mb[k][1]
        pltpu.sync_copy(sidx_hbm.at[pl.ds(wbase + gi * CH, CH)], sidx_v)
        return [
            pltpu.async_copy(nb_hbm.at[sidx_v], nbv, msem[k]),
            pltpu.async_copy(v0_hbm.at[sidx_v], mb[k][5], msem[k]),
            pltpu.async_copy(v1_hbm.at[sidx_v], mb[k][6], msem[k]),
            pltpu.async_copy(v2_hbm.at[sidx_v], mb[k][7], msem[k]),
            pltpu.async_copy(v3_hbm.at[sidx_v], mb[k][8], msem[k]),
        ]

    def start_rows(k, rk):
        nbv, i1, i2, i3 = mb[k][1], mb[k][2], mb[k][3], mb[k][4]
        for g in range(GRP):
            s = pl.ds(g * 16, 16)
            v = nbv[s]
            i1[s] = v + 1
            i2[s] = v + W
            i3[s] = v + (W + 1)
        return [
            pltpu.async_copy(xt_hbm.at[nbv], rb[rk][0], rsem[rk]),
            pltpu.async_copy(xt_hbm.at[i1], rb[rk][1], rsem[rk]),
            pltpu.async_copy(xt_hbm.at[i2], rb[rk][2], rsem[rk]),
            pltpu.async_copy(xt_hbm.at[i3], rb[rk][3], rsem[rk]),
        ]

    def blend(k, rk):
        w0, w1, w2, w3 = mb[k][5], mb[k][6], mb[k][7], mb[k][8]
        r0, r1, r2, r3 = rb[rk]
        obk = ob[rk]

        def tok_body(g, c2):
            base = g * 16
            aw0 = w0[pl.ds(base, 16)]
            aw1 = w1[pl.ds(base, 16)]
            aw2 = w2[pl.ds(base, 16)]
            aw3 = w3[pl.ds(base, 16)]
            for l in range(16):
                t = base + l
                a0 = aw0[l]
                a1 = aw1[l]
                a2 = aw2[l]
                a3 = aw3[l]
                for v in range(C // 16):
                    s = pl.ds(v * 16, 16)
                    obk[t, s] = r0[t, s] * a0 + r1[t, s] * a1 \
                        + r2[t, s] * a2 + r3[t, s] * a3
            return c2

        lax.fori_loop(0, GRP, tok_body, 0)

    def body(grp_i, carry):
        g0 = grp_i * UNROLL

        metacps = [None] * UNROLL
        rowcps = [None] * UNROLL
        outcps = [None] * UNROLL
        for i in range(min(3, UNROLL)):
            metacps[i] = start_meta(g0 + i, i % NMB)
        for cp in metacps[0]:
            cp.wait()
        rowcps[0] = start_rows(0, 0)

        for i in range(UNROLL):
            if i + 3 < UNROLL:
                metacps[i + 3] = start_meta(g0 + i + 3, (i + 3) % NMB)
            if i + 1 < UNROLL:
                for cp in metacps[i + 1]:
                    cp.wait()
                rowcps[i + 1] = start_rows((i + 1) % NMB, (i + 1) % NRB)
            for cp in rowcps[i]:
                cp.wait()
            if i >= NRB:
                outcps[i - NRB].wait()
            blend(i % NMB, i % NRB)
            outcps[i] = pltpu.async_copy(
                ob[i % NRB], out_hbm.at[pl.ds(wbase + (g0 + i) * CH, CH)],
                osem[i % NRB])

        for i in range(UNROLL - NRB, UNROLL):
            outcps[i].wait()
        return carry

    lax.fori_loop(0, NCH // UNROLL, body, 0)


def kernel(x, delta_p, delta_t):
    b, c, h, w = x.shape
    hw = h * w
    n = b * hw

    gyy, gxx = jnp.meshgrid(jnp.linspace(-1.0, 1.0, h),
                            jnp.linspace(-1.0, 1.0, w), indexing="ij")
    gx = gxx[None] + delta_p[:, 0]
    gy = gyy[None] + delta_p[:, 1]
    ix = ((gx + 1.0) * w - 1.0) / 2.0
    iy = ((gy + 1.0) * h - 1.0) / 2.0
    ix0 = jnp.floor(ix)
    iy0 = jnp.floor(iy)
    ix1 = ix0 + 1.0
    iy1 = iy0 + 1.0
    wx1 = ix - ix0
    wy1 = iy - iy0
    wx0 = 1.0 - wx1
    wy0 = 1.0 - wy1

    def wcorner(ixq, iyq, wt):
        valid = (ixq >= 0.0) & (ixq <= w - 1.0) & (iyq >= 0.0) & (iyq <= h - 1.0)
        return jnp.where(valid, wt, 0.0).reshape(n)

    w00 = wcorner(ix0, iy0, wx0 * wy0)
    w01 = wcorner(ix1, iy0, wx1 * wy0)
    w10 = wcorner(ix0, iy1, wx0 * wy1)
    w11 = wcorner(ix1, iy1, wx1 * wy1)

    boff = (jnp.arange(b, dtype=jnp.int32) * hw)[:, None, None]
    iy0c = jnp.clip(iy0, -1.0, h - 1.0).astype(jnp.int32)
    ix0c = jnp.clip(ix0, -1.0, w - 1.0).astype(jnp.int32)
    nb = (iy0c * w + ix0c + boff + PAD).reshape(n)

    ref_idx = (jnp.arange(hw, dtype=jnp.float32).reshape(1, 1, h, w)
               / (hw - 1) * 2.0 - 1.0)
    keys = (ref_idx + delta_t).reshape(b, hw)
    sidx = jnp.argsort(keys, axis=1).astype(jnp.int32)
    sidx = (sidx + (jnp.arange(b, dtype=jnp.int32) * hw)[:, None]).reshape(n)

    xt = jnp.transpose(x.reshape(b, c, hw), (0, 2, 1)).reshape(n, c)
    xtp = jnp.pad(xt, ((PAD, PAD), (0, 0)))

    out = _sc_gather(xtp, nb, w00, w01, w10, w11, sidx)
    return out.reshape(b, hw, c)
